# baseline (speedup 1.0000x reference)
import functools

import jax
import jax.numpy as jnp
from jax import lax
from jax.experimental import pallas as pl
from jax.experimental.pallas import tpu as pltpu
from jax.experimental.pallas import tpu_sc as plsc

_B, _H, _W, _C = 4, 192, 192, 96
_HO, _WO = 2 * _H, 2 * _W
_INB = _H * _W * _C
_OUTB = _HO * _WO * _C
_OUT = _B * _OUTB

_NC, _NS, _L = 2, 16, 16
_BPC = _B // _NC
_SH = _BPC * _INB // _NS
_S = 16
_CH = _BPC * _OUTB // _S
_K = 1728
_NBLK = _SH // _K
_NB2 = _NBLK // 2
_NG = _K // _L
_GPC = 12
_NCH = _NG // _GPC
_PB = 1728
_PBW = _PB + _GPC * _L
_PGR = _PBW // _L
_ZS = _CH // _NS
_ZK = 9216
_NZ = _ZS // _ZK
_WPB = _INB // _SH

_mesh = plsc.VectorSubcoreMesh(
    core_axis_name="c", subcore_axis_name="s",
    num_cores=_NC, num_subcores=_NS)


_N = _B * _INB


@functools.partial(
    pl.kernel,
    out_type=(jax.ShapeDtypeStruct((_OUT,), jnp.float32),
              jax.ShapeDtypeStruct((_N,), jnp.int32)),
    mesh=_mesh,
    compiler_params=pltpu.CompilerParams(needs_layout_passes=False),
    scratch_types=[
        pltpu.VMEM((_K,), jnp.int32),
        pltpu.VMEM((_K,), jnp.float32),
        pltpu.VMEM((_K,), jnp.int32),
        pltpu.VMEM((_K,), jnp.float32),
        pltpu.VMEM((_PBW,), jnp.int32),
        pltpu.VMEM((_PBW,), jnp.float32),
        pltpu.VMEM((_ZK,), jnp.float32),
        pltpu.VMEM_SHARED((_CH,), jnp.float32),
        pltpu.SemaphoreType.DMA,
        pltpu.SemaphoreType.DMA,
        pltpu.SemaphoreType.DMA,
    ],
)
def _unpool(mask_hbm, upd_hbm, out_hbm, g_hbm,
            mba, uba, mbb, ubb, pidx, pval, zbuf, acc,
            sin_a, sin_b, sz):
  ci = lax.axis_index("c")
  si = lax.axis_index("s")

  def zinit(i, carry):
    zbuf[pl.ds(i * _L, _L)] = jnp.zeros((_L,), jnp.float32)
    return carry
  lax.fori_loop(0, _ZK // _L, zinit, 0)

  in_base = ci * (_BPC * _INB) + si * _SH
  out_base = ci * (_BPC * _OUTB)
  gbase = (ci * _BPC + si // _WPB) * _OUTB

  def start_in(blk, mb, ub, sem):
    base = in_base + blk * _K
    pltpu.async_copy(g_hbm.at[pl.ds(base, _K)], mb, sem)
    pltpu.async_copy(upd_hbm.at[pl.ds(base, _K)], ub, sem)

  def wait_in(blk, mb, ub, sem):
    base = in_base + blk * _K
    pltpu.make_async_copy(g_hbm.at[pl.ds(base, _K)], mb, sem).wait()
    pltpu.make_async_copy(upd_hbm.at[pl.ds(base, _K)], ub, sem).wait()

  def start_mask(blk, mb, sem):
    pltpu.async_copy(mask_hbm.at[pl.ds(in_base + blk * _K, _K)], mb, sem)

  def wait_mask(blk, mb, sem):
    pltpu.make_async_copy(
        mask_hbm.at[pl.ds(in_base + blk * _K, _K)], mb, sem).wait()

  def pre_decode(mb):
    def chunk(t, carry):
      for u in range(_GPC):
        j = t * _GPC + u
        m = mb[pl.ds(j * _L, _L)]
        cv = jnp.full((_L,), _C, jnp.int32)
        third = jnp.full((_L,), jnp.float32(1.0 / 3.0))
        q = (lax.shift_right_logical(m, jnp.full((_L,), 5, jnp.int32))
             .astype(jnp.float32) * third).astype(jnp.int32)
        off = jnp.full((_L,), (u % 6) * _L + gbase - out_base,
                       jnp.int32) + lax.iota(jnp.int32, _L)
        pidx[pl.ds(j * _L, _L)] = q * cv + off
      return carry
    lax.fori_loop(0, _NCH, chunk, 0)

  start_mask(0, mba, sin_a)
  start_mask(1, mbb, sin_b)

  def pre(i, carry):
    wait_mask(2 * i, mba, sin_a)
    pre_decode(mba)
    pltpu.sync_copy(pidx.at[pl.ds(0, _K)],
                    g_hbm.at[pl.ds(in_base + 2 * i * _K, _K)])

    @pl.when(i < _NB2 - 1)
    def _():
      start_mask(2 * i + 2, mba, sin_a)

    wait_mask(2 * i + 1, mbb, sin_b)
    pre_decode(mbb)
    pltpu.sync_copy(pidx.at[pl.ds(0, _K)],
                    g_hbm.at[pl.ds(in_base + (2 * i + 1) * _K, _K)])

    @pl.when(i < _NB2 - 1)
    def _():
      start_mask(2 * i + 3, mbb, sin_b)
    return carry
  lax.fori_loop(0, _NB2, pre, 0)

  def flush(basev):
    pos = basev[0]

    def clr(t, carry):
      old = pidx[pl.ds(t * _L, _L)]
      keep = (jnp.full((_L,), t * _L, jnp.int32) + lax.iota(jnp.int32, _L)
              ) < jnp.full((_L,), pos, jnp.int32)
      pidx[pl.ds(t * _L, _L)] = jnp.where(
          keep, old, jnp.full((_L,), -1, jnp.int32))
      return carry
    lax.fori_loop(pos // _L, _PGR, clr, 0)
    pltpu.sync_copy(
        pval, acc.at[plsc.Indices(pidx, ignored_value=-1)], add=True)
    return jnp.full((_L,), 0, jnp.int32)

  def consume(mb, ub, lo, basev):
    chv = jnp.full((_L,), _CH, jnp.uint32)
    onev = jnp.full((_L,), 1, jnp.int32)
    zerov = jnp.full((_L,), 0, jnp.int32)

    def chunk(t, basev):
      for u in range(_GPC):
        j = t * _GPC + u
        g = mb[pl.ds(j * _L, _L)]
        v = ub[pl.ds(j * _L, _L)]
        li = g - jnp.full((_L,), lo - out_base, jnp.int32)
        mk = plsc.bitcast(li, jnp.uint32) < chv
        ones = jnp.where(mk, onev, zerov)
        cs = plsc.cumsum(ones)
        dest = basev + (cs - ones)
        plsc.store_scatter(pidx, [dest], li, mask=mk)
        plsc.store_scatter(pval, [dest], v, mask=mk)
        basev = basev + plsc.all_reduce_population_count(mk)
      return lax.cond(basev[0] >= _PB, flush, lambda b: b, basev)
    return lax.fori_loop(0, _NCH, chunk, basev)

  def sweep(s, scarry):
    lo = out_base + s * _CH

    def zstart(t, carry):
      pltpu.async_copy(zbuf, acc.at[pl.ds(si * _ZS + t * _ZK, _ZK)], sz)
      return carry
    lax.fori_loop(0, _NZ, zstart, 0)

    def zwait(t, carry):
      pltpu.make_async_copy(
          zbuf, acc.at[pl.ds(si * _ZS + t * _ZK, _ZK)], sz).wait()
      return carry
    lax.fori_loop(0, _NZ, zwait, 0)
    plsc.subcore_barrier()

    start_in(0, mba, uba, sin_a)
    start_in(1, mbb, ubb, sin_b)

    def pipe(i, basev):
      wait_in(2 * i, mba, uba, sin_a)
      basev = consume(mba, uba, lo, basev)

      @pl.when(i < _NB2 - 1)
      def _():
        start_in(2 * i + 2, mba, uba, sin_a)

      wait_in(2 * i + 1, mbb, ubb, sin_b)
      basev = consume(mbb, ubb, lo, basev)

      @pl.when(i < _NB2 - 1)
      def _():
        start_in(2 * i + 3, mbb, ubb, sin_b)
      return basev
    basev = lax.fori_loop(0, _NB2, pipe, jnp.full((_L,), 0, jnp.int32))
    flush(basev)
    plsc.subcore_barrier()

    pltpu.sync_copy(acc.at[pl.ds(si * _ZS, _ZS)],
                    out_hbm.at[pl.ds(lo + si * _ZS, _ZS)])
    plsc.subcore_barrier()
    return scarry

  lax.fori_loop(0, _S, sweep, 0)


def kernel(updates, mask):
  m = mask.astype(jnp.int32).reshape(-1)
  u = updates.reshape(-1)
  out, _ = _unpool(m, u)
  return out.reshape(_B, _HO, _WO, _C)


# speedup vs baseline: 2.1388x; 2.1388x over previous
import functools

import jax
import jax.numpy as jnp
from jax import lax
from jax.experimental import pallas as pl
from jax.experimental.pallas import tpu as pltpu
from jax.experimental.pallas import tpu_sc as plsc

_B, _H, _W, _C = 4, 192, 192, 96
_HO, _WO = 2 * _H, 2 * _W
_INB = _H * _W * _C
_OUTB = _HO * _WO * _C
_OUT = _B * _OUTB

_NC, _NS, _L = 2, 16, 16
_BPC = _B // _NC
_SH = _BPC * _INB // _NS
_S = 16
_CH = _BPC * _OUTB // _S
_K = 1728
_NBLK = _SH // _K
_NB2 = _NBLK // 2
_NG = _K // _L
_GPC = 12
_NCH = _NG // _GPC
_PB = 1728
_PBW = _PB + _GPC * _L
_PGR = _PBW // _L
_ZS = _CH // _NS
_ZK = 9216
_NZ = _ZS // _ZK
_WPB = _INB // _SH

_mesh = plsc.VectorSubcoreMesh(
    core_axis_name="c", subcore_axis_name="s",
    num_cores=_NC, num_subcores=_NS)


@functools.partial(
    pl.kernel,
    out_type=jax.ShapeDtypeStruct((_OUT,), jnp.float32),
    mesh=_mesh,
    compiler_params=pltpu.CompilerParams(needs_layout_passes=False),
    scratch_types=[
        pltpu.VMEM((_K,), jnp.int32),
        pltpu.VMEM((_K,), jnp.float32),
        pltpu.VMEM((_K,), jnp.int32),
        pltpu.VMEM((_K,), jnp.float32),
        pltpu.VMEM((_PBW,), jnp.int32),
        pltpu.VMEM((_PBW,), jnp.float32),
        pltpu.VMEM((_ZK,), jnp.float32),
        pltpu.VMEM_SHARED((_CH,), jnp.float32),
        pltpu.SemaphoreType.DMA,
        pltpu.SemaphoreType.DMA,
        pltpu.SemaphoreType.DMA,
    ],
)
def _unpool(mask_hbm, upd_hbm, out_hbm,
            mba, uba, mbb, ubb, pidx, pval, zbuf, acc,
            sin_a, sin_b, sz):
  ci = lax.axis_index("c")
  si = lax.axis_index("s")

  def zinit(i, carry):
    zbuf[pl.ds(i * _L, _L)] = jnp.zeros((_L,), jnp.float32)
    return carry
  lax.fori_loop(0, _ZK // _L, zinit, 0)

  in_base = ci * (_BPC * _INB) + si * _SH
  out_base = ci * (_BPC * _OUTB)
  gbase = (ci * _BPC + si // _WPB) * _OUTB

  def start_in(blk, mb, ub, sem):
    base = in_base + blk * _K
    pltpu.async_copy(mask_hbm.at[pl.ds(base, _K)], mb, sem)
    pltpu.async_copy(upd_hbm.at[pl.ds(base, _K)], ub, sem)

  def wait_in(blk, mb, ub, sem):
    base = in_base + blk * _K
    pltpu.make_async_copy(mask_hbm.at[pl.ds(base, _K)], mb, sem).wait()
    pltpu.make_async_copy(upd_hbm.at[pl.ds(base, _K)], ub, sem).wait()

  def flush(pos):
    def clr(t, carry):
      old = pidx[pl.ds(t * _L, _L)]
      keep = (jnp.full((_L,), t * _L, jnp.int32) + lax.iota(jnp.int32, _L)
              ) < jnp.full((_L,), pos, jnp.int32)
      pidx[pl.ds(t * _L, _L)] = jnp.where(
          keep, old, jnp.full((_L,), -1, jnp.int32))
      return carry
    lax.fori_loop(pos // _L, _PGR, clr, 0)
    pltpu.sync_copy(
        pval, acc.at[plsc.Indices(pidx, ignored_value=-1)], add=True)
    return jnp.int32(0)

  def consume(mb, ub, lo, pos):
    chv = jnp.full((_L,), _CH, jnp.uint32)

    def chunk(t, pos):
      lis, vs, cnts = [], [], []
      for u in range(_GPC):
        j = t * _GPC + u
        m = mb[pl.ds(j * _L, _L)]
        v = ub[pl.ds(j * _L, _L)]
        cv = jnp.full((_L,), _C, jnp.int32)
        third = jnp.full((_L,), jnp.float32(1.0 / 3.0))
        q = (lax.shift_right_logical(m, jnp.full((_L,), 5, jnp.int32))
             .astype(jnp.float32) * third).astype(jnp.int32)
        off = jnp.full((_L,), (u % 6) * _L + gbase - lo,
                       jnp.int32) + lax.iota(jnp.int32, _L)
        li = q * cv + off
        mk = plsc.bitcast(li, jnp.uint32) < chv
        lis.append(li)
        vs.append(v)
        cnts.append(plsc.all_reduce_population_count(mk)[0])
      base = pos
      for u in range(_GPC):
        mk = plsc.bitcast(lis[u], jnp.uint32) < chv
        plsc.store_compressed(pidx.at[pl.ds(base, _L)], lis[u], mask=mk)
        plsc.store_compressed(pval.at[pl.ds(base, _L)], vs[u], mask=mk)
        base = base + cnts[u]
      return lax.cond(base >= _PB, flush, lambda p: p, base)
    return lax.fori_loop(0, _NCH, chunk, pos)

  def sweep(s, scarry):
    lo = out_base + s * _CH

    def zstart(t, carry):
      pltpu.async_copy(zbuf, acc.at[pl.ds(si * _ZS + t * _ZK, _ZK)], sz)
      return carry
    lax.fori_loop(0, _NZ, zstart, 0)

    def zwait(t, carry):
      pltpu.make_async_copy(
          zbuf, acc.at[pl.ds(si * _ZS + t * _ZK, _ZK)], sz).wait()
      return carry
    lax.fori_loop(0, _NZ, zwait, 0)
    plsc.subcore_barrier()

    start_in(0, mba, uba, sin_a)
    start_in(1, mbb, ubb, sin_b)

    def pipe(i, pos):
      wait_in(2 * i, mba, uba, sin_a)
      pos = consume(mba, uba, lo, pos)

      @pl.when(i < _NB2 - 1)
      def _():
        start_in(2 * i + 2, mba, uba, sin_a)

      wait_in(2 * i + 1, mbb, ubb, sin_b)
      pos = consume(mbb, ubb, lo, pos)

      @pl.when(i < _NB2 - 1)
      def _():
        start_in(2 * i + 3, mbb, ubb, sin_b)
      return pos
    pos = lax.fori_loop(0, _NB2, pipe, jnp.int32(0))
    flush(pos)
    plsc.subcore_barrier()

    pltpu.sync_copy(acc.at[pl.ds(si * _ZS, _ZS)],
                    out_hbm.at[pl.ds(lo + si * _ZS, _ZS)])
    plsc.subcore_barrier()
    return scarry

  lax.fori_loop(0, _S, sweep, 0)


def kernel(updates, mask):
  m = mask.astype(jnp.int32).reshape(-1)
  u = updates.reshape(-1)
  out = _unpool(m, u)
  return out.reshape(_B, _HO, _WO, _C)
